# Initial kernel scaffold; baseline (speedup 1.0000x reference)
import jax
import jax.numpy as jnp
from jax.experimental import pallas as pl


def kernel(indices, table):
    raise NotImplementedError("write your pallas kernel here")


# speedup vs baseline: 1.6852x; 1.6852x over previous
import functools

import jax
import jax.numpy as jnp
from jax import lax
from jax.experimental import pallas as pl
from jax.experimental.pallas import tpu as pltpu
from jax.experimental.pallas import tpu_sc as plsc

EMB = 64

_info = plsc.get_sparse_core_info()
_NC = _info.num_cores
_NS = _info.num_subcores
_NW = _NC * _NS

CHUNK = 128


def _sc_gather(idx, table):
    B = idx.shape[0]
    assert B % (_NW * CHUNK) == 0
    b_per_w = B // _NW
    n_chunks = b_per_w // CHUNK
    idx3 = idx.reshape(_NW, n_chunks, CHUNK)

    mesh = plsc.VectorSubcoreMesh(core_axis_name="c", subcore_axis_name="s")

    @functools.partial(
        pl.kernel,
        mesh=mesh,
        out_type=jax.ShapeDtypeStruct((B, EMB), jnp.float32),
        compiler_params=pltpu.CompilerParams(use_tc_tiling_on_sc=False),
        scratch_types=[
            pltpu.VMEM((n_chunks, CHUNK), jnp.int32),
            pltpu.VMEM((CHUNK, EMB), jnp.float32),
            pltpu.SemaphoreType.DMA,
        ],
    )
    def k(idx_hbm, table_hbm, out_hbm, idx_v, rows_v, sem):
        wid = lax.axis_index("s") * _NC + lax.axis_index("c")
        base = wid * b_per_w
        pltpu.sync_copy(idx_hbm.at[wid], idx_v)

        def body(ci, carry):
            pltpu.async_copy(table_hbm.at[idx_v.at[ci]], rows_v, sem).wait()
            pltpu.sync_copy(rows_v, out_hbm.at[pl.ds(base + ci * CHUNK, CHUNK)])
            return carry

        lax.fori_loop(0, n_chunks, body, 0)

    return k(idx3, table)


def kernel(indices, table):
    idx = indices.reshape(-1).astype(jnp.int32)
    out = _sc_gather(idx, table)
    return out.reshape(indices.shape + (EMB,))


# speedup vs baseline: 1.8771x; 1.1139x over previous
import functools

import jax
import jax.numpy as jnp
from jax import lax
from jax.experimental import pallas as pl
from jax.experimental.pallas import tpu as pltpu
from jax.experimental.pallas import tpu_sc as plsc

EMB = 64

_info = plsc.get_sparse_core_info()
_NC = _info.num_cores
_NS = _info.num_subcores
_NW = _NC * _NS

CHUNK = 128
NBUF = 8


def _sc_gather(idx, table):
    B = idx.shape[0]
    assert B % (_NW * CHUNK * NBUF) == 0
    b_per_w = B // _NW
    n_chunks = b_per_w // CHUNK
    n_outer = n_chunks // NBUF
    idx3 = idx.reshape(_NW, n_chunks, CHUNK)

    mesh = plsc.VectorSubcoreMesh(core_axis_name="c", subcore_axis_name="s")

    scratch = (
        [pltpu.VMEM((n_chunks, CHUNK), jnp.int32)]
        + [pltpu.VMEM((CHUNK, EMB), jnp.float32) for _ in range(NBUF)]
        + [pltpu.SemaphoreType.DMA for _ in range(2 * NBUF)]
    )

    @functools.partial(
        pl.kernel,
        mesh=mesh,
        out_type=jax.ShapeDtypeStruct((B, EMB), jnp.float32),
        compiler_params=pltpu.CompilerParams(use_tc_tiling_on_sc=False),
        scratch_types=scratch,
    )
    def k(idx_hbm, table_hbm, out_hbm, idx_v, *bufs_and_sems):
        rows = bufs_and_sems[:NBUF]
        gsem = bufs_and_sems[NBUF : 2 * NBUF]
        osem = bufs_and_sems[2 * NBUF : 3 * NBUF]

        wid = lax.axis_index("s") * _NC + lax.axis_index("c")
        base = wid * b_per_w
        pltpu.sync_copy(idx_hbm.at[wid], idx_v)

        def gather_start(ci, b):
            pltpu.async_copy(table_hbm.at[idx_v.at[ci]], rows[b], gsem[b])

        def gather_wait(ci, b):
            pltpu.make_async_copy(
                table_hbm.at[idx_v.at[ci]], rows[b], gsem[b]
            ).wait()

        def write_start(ci, b):
            pltpu.async_copy(
                rows[b], out_hbm.at[pl.ds(base + ci * CHUNK, CHUNK)], osem[b]
            )

        def write_wait(ci, b):
            pltpu.make_async_copy(
                rows[b], out_hbm.at[pl.ds(base + ci * CHUNK, CHUNK)], osem[b]
            ).wait()

        for b in range(NBUF):
            gather_start(b, b)

        def outer(o, carry):
            for b in range(NBUF):
                c = o * NBUF + b
                gather_wait(c, b)
                write_start(c, b)
                write_wait(c, b)
                gather_start(c + NBUF, b)
            return carry

        lax.fori_loop(0, n_outer - 1, outer, 0)

        for b in range(NBUF):
            c = (n_outer - 1) * NBUF + b
            gather_wait(c, b)
            write_start(c, b)
        for b in range(NBUF):
            c = (n_outer - 1) * NBUF + b
            write_wait(c, b)

    return k(idx3, table)


def kernel(indices, table):
    idx = indices.reshape(-1).astype(jnp.int32)
    out = _sc_gather(idx, table)
    return out.reshape(indices.shape + (EMB,))
